# baseline (speedup 1.0000x reference)
import functools
import jax
import jax.numpy as jnp
from jax import lax
from jax.experimental import pallas as pl
from jax.experimental.pallas import tpu as pltpu
from jax.experimental.pallas import tpu_sc as plsc

D = 64
B_ROWS = 16384 * 50
NC, NS = 2, 16
NW = NC * NS
CHUNK = 128
PER_W = B_ROWS // NW
NCH = PER_W // CHUNK
SCALE = 8.0


def _sc_embed(x2d, table):
    mesh = plsc.VectorSubcoreMesh(core_axis_name="c", subcore_axis_name="s")

    @functools.partial(
        pl.kernel,
        out_type=jax.ShapeDtypeStruct((B_ROWS, D), jnp.float32),
        mesh=mesh,
        scratch_types=[
            pltpu.VMEM((NCH, CHUNK), jnp.int32),
            pltpu.VMEM((CHUNK, D), jnp.float32),
            pltpu.SemaphoreType.DMA,
        ],
        compiler_params=pltpu.CompilerParams(use_tc_tiling_on_sc=False),
    )
    def k(x_hbm, table_hbm, out_hbm, idx_v, rows_v, sem):
        wid = lax.axis_index("s") * NC + lax.axis_index("c")
        pltpu.sync_copy(x_hbm.at[pl.ds(wid * NCH, NCH)], idx_v)

        def chunk_body(j, carry):
            pltpu.async_copy(table_hbm.at[idx_v.at[j]], rows_v, sem).wait()

            def row_body(i, c2):
                for t in range(D // 16):
                    rows_v[i, pl.ds(t * 16, 16)] = (
                        rows_v[i, pl.ds(t * 16, 16)] * SCALE
                    )
                return c2

            lax.fori_loop(0, CHUNK, row_body, 0)
            pltpu.sync_copy(
                rows_v, out_hbm.at[pl.ds(wid * PER_W + j * CHUNK, CHUNK)]
            )
            return carry

        lax.fori_loop(0, NCH, chunk_body, 0)

    return k(x2d, table)


def kernel(x, table):
    x2d = x.reshape(NW * NCH, CHUNK).astype(jnp.int32)
    out = _sc_embed(x2d, table)
    return out.reshape(x.shape[0], x.shape[1], D)


# speedup vs baseline: 1.1814x; 1.1814x over previous
import functools
import jax
import jax.numpy as jnp
from jax import lax
from jax.experimental import pallas as pl
from jax.experimental.pallas import tpu as pltpu
from jax.experimental.pallas import tpu_sc as plsc

D = 64
B_ROWS = 16384 * 50
NC, NS = 2, 16
NW = NC * NS
CHUNK = 128
PER_W = B_ROWS // NW
NCH = PER_W // CHUNK
NBUF = 2
ROW_UNROLL = 8
SCALE = 8.0


def _sc_embed(x2d, table):
    mesh = plsc.VectorSubcoreMesh(core_axis_name="c", subcore_axis_name="s")

    @functools.partial(
        pl.kernel,
        out_type=jax.ShapeDtypeStruct((B_ROWS, D), jnp.float32),
        mesh=mesh,
        scratch_types=[
            pltpu.VMEM((NCH, CHUNK), jnp.int32),
            pltpu.VMEM((CHUNK, D), jnp.float32),
            pltpu.VMEM((CHUNK, D), jnp.float32),
            pltpu.VMEM((CHUNK, D), jnp.float32),
            pltpu.VMEM((CHUNK, D), jnp.float32),
            pltpu.SemaphoreType.DMA,
            pltpu.SemaphoreType.DMA,
            pltpu.SemaphoreType.DMA,
            pltpu.SemaphoreType.DMA,
        ],
        compiler_params=pltpu.CompilerParams(use_tc_tiling_on_sc=False),
    )
    def k(x_hbm, table_hbm, out_hbm, idx_v, g0, g1, w0, w1,
          gs0, gs1, ws0, ws1):
        gbuf, wbuf = [g0, g1], [w0, w1]
        gsem, wsem = [gs0, gs1], [ws0, ws1]
        wid = lax.axis_index("s") * NC + lax.axis_index("c")
        row0 = wid * PER_W

        pltpu.sync_copy(x_hbm.at[pl.ds(wid * NCH, NCH)], idx_v)

        for b in range(NBUF):
            pltpu.async_copy(table_hbm.at[idx_v.at[b]], gbuf[b], gsem[b])

        def scale_chunk(gb, wb):
            def sbody(i, c):
                base = i * ROW_UNROLL
                for u in range(ROW_UNROLL):
                    for t in range(D // 16):
                        sl = pl.ds(t * 16, 16)
                        wb[base + u, sl] = gb[base + u, sl] * SCALE
                return c
            lax.fori_loop(0, CHUNK // ROW_UNROLL, sbody, 0)

        def outer(g, carry):
            for b in range(NBUF):
                j = g * NBUF + b
                pltpu.make_async_copy(
                    table_hbm.at[idx_v.at[j]], gbuf[b], gsem[b]
                ).wait()
                @pl.when(j >= NBUF)
                def _():
                    pltpu.make_async_copy(
                        wbuf[b],
                        out_hbm.at[pl.ds(row0 + (j - NBUF) * CHUNK, CHUNK)],
                        wsem[b],
                    ).wait()

                scale_chunk(gbuf[b], wbuf[b])
                pltpu.async_copy(
                    wbuf[b],
                    out_hbm.at[pl.ds(row0 + j * CHUNK, CHUNK)],
                    wsem[b],
                )

                @pl.when(j + NBUF < NCH)
                def _():
                    pltpu.async_copy(
                        table_hbm.at[idx_v.at[j + NBUF]], gbuf[b], gsem[b]
                    )
            return carry

        lax.fori_loop(0, NCH // NBUF, outer, 0)

        for b in range(NBUF):
            j = NCH - NBUF + b
            pltpu.make_async_copy(
                wbuf[b],
                out_hbm.at[pl.ds(row0 + j * CHUNK, CHUNK)],
                wsem[b],
            ).wait()

    return k(x2d, table)


def kernel(x, table):
    x2d = x.reshape(NW * NCH, CHUNK).astype(jnp.int32)
    out = _sc_embed(x2d, table)
    return out.reshape(x.shape[0], x.shape[1], D)
